# baseline (speedup 1.0000x reference)
import functools
import jax
import jax.numpy as jnp
from jax import lax
from jax.experimental import pallas as pl
from jax.experimental.pallas import tpu as pltpu
from jax.experimental.pallas import tpu_sc as plsc

VOCAB = 50257
N_POS = 1024
D = 768
B = 64
S = 1024

NC = 2
NS = 16
NW = NC * NS
LANES = 16
D_SLICES = D // LANES

QB = 4
BL = B // QB
ST = 128

NB = 8
DEPTH = 4
SC_W = 16
KC = ST // SC_W
STEPS = KC * BL
UNROLL = NB


def _body(ids_hbm, wte_hbm, wpe_hbm, out_hbm, idx_v, wpe_v, *scratch):
    rows = scratch[:NB]
    gsem = scratch[NB:2 * NB]
    ssem = scratch[2 * NB:3 * NB]

    cid = lax.axis_index("c")
    sid = lax.axis_index("s")
    wid = sid * NC + cid
    t = wid // QB
    q = wid % QB
    s_base = t * ST

    pltpu.sync_copy(ids_hbm.at[pl.ds(q * BL, BL), pl.ds(t * ST, ST)], idx_v)

    def idx_slice(i):
        k = lax.shift_right_logical(i, 4)
        bb = lax.bitwise_and(i, 15)
        return idx_v.at[bb, pl.ds(k * SC_W, SC_W)]

    def out_slice(i):
        k = lax.shift_right_logical(i, 4)
        bb = lax.bitwise_and(i, 15)
        return out_hbm.at[q * BL + bb, pl.ds(s_base + k * SC_W, SC_W), :]

    def issue_gather(i, slot):
        pltpu.async_copy(wte_hbm.at[idx_slice(i)], rows[slot], gsem[slot])

    def wait_gather(i, slot):
        pltpu.make_async_copy(
            wte_hbm.at[idx_slice(i)], rows[slot], gsem[slot]).wait()

    def issue_store(i, slot):
        pltpu.async_copy(rows[slot], out_slice(i), ssem[slot])

    def wait_store(i, slot):
        pltpu.make_async_copy(rows[slot], out_slice(i), ssem[slot]).wait()

    def handle_wpe(i):
        k = lax.shift_right_logical(i, 4)
        bb = lax.bitwise_and(i, 15)

        @pl.when(bb == 0)
        def _():
            pltpu.sync_copy(wpe_hbm.at[pl.ds(s_base + k * SC_W, SC_W), :],
                            wpe_v)

    def add_wpe(i, slot):
        @plsc.parallel_loop(0, SC_W, 1, unroll=1)
        def _(r):
            for c in range(D_SLICES):
                sl = pl.ds(c * LANES, LANES)
                plsc.addupdate(rows[slot].at[r, sl], wpe_v[r, sl])

    def step(i, slot):
        far = (slot + DEPTH) % NB

        @pl.when(i >= DEPTH)
        def _():
            wait_store(i - DEPTH, far)

        @pl.when(i + DEPTH < STEPS)
        def _():
            issue_gather(i + DEPTH, far)
        wait_gather(i, slot)
        handle_wpe(i)
        add_wpe(i, slot)
        issue_store(i, slot)

    for s0 in range(DEPTH):
        issue_gather(jnp.int32(s0), s0)

    def loop_body(j, _):
        i = UNROLL * j
        for r in range(UNROLL):
            step(i + r, r)
        return _

    lax.fori_loop(0, STEPS // UNROLL, loop_body, None)
    for d in range(DEPTH):
        i = STEPS - DEPTH + d
        wait_store(i, i % NB)


@jax.jit
def _embed(input_ids, wte, wpe):
    mesh = plsc.VectorSubcoreMesh(core_axis_name="c", subcore_axis_name="s")
    return pl.kernel(
        _body,
        out_type=jax.ShapeDtypeStruct((B, S, D), jnp.float32),
        mesh=mesh,
        scratch_types=(
            [pltpu.VMEM((BL, ST), jnp.int32),
             pltpu.VMEM((SC_W, D), jnp.float32)]
            + [pltpu.VMEM((SC_W, D), jnp.float32)] * NB
            + [pltpu.SemaphoreType.DMA] * (2 * NB)
        ),
    )(input_ids, wte, wpe)


def kernel(input_ids, attention_mask, wte, wpe):
    del attention_mask
    return _embed(input_ids, wte, wpe)


# speedup vs baseline: 1.0689x; 1.0689x over previous
import functools
import jax
import jax.numpy as jnp
from jax import lax
from jax.experimental import pallas as pl
from jax.experimental.pallas import tpu as pltpu
from jax.experimental.pallas import tpu_sc as plsc

VOCAB = 50257
N_POS = 1024
D = 768
B = 64
S = 1024

NC = 2
NS = 16
NW = NC * NS
LANES = 16
D_SLICES = D // LANES

SC_W = S // NW
NB = 4
DEPTH = 2
STEPS = B
UNROLL = NB


def _body(ids_hbm, wte_hbm, wpe_hbm, out_hbm, idx_v, wpe_v, *scratch):
    rows = scratch[:NB]
    gsem = scratch[NB:2 * NB]
    ssem = scratch[2 * NB:3 * NB]
    isem = scratch[3 * NB]

    cid = lax.axis_index("c")
    sid = lax.axis_index("s")
    wid = sid * NC + cid
    s0 = wid * SC_W

    for b in range(B):
        pltpu.async_copy(ids_hbm.at[pl.ds(b * S + s0, SC_W)], idx_v.at[b],
                         isem)
    for b in range(B):
        pltpu.make_async_copy(ids_hbm.at[pl.ds(b * S + s0, SC_W)],
                              idx_v.at[b], isem).wait()
    pltpu.sync_copy(wpe_hbm.at[pl.ds(s0, SC_W), :], wpe_v)

    def idx_slice(i):
        return idx_v.at[i]

    def out_slice(i):
        return out_hbm.at[i, pl.ds(s0, SC_W), :]

    def issue_gather(i, slot):
        pltpu.async_copy(wte_hbm.at[idx_slice(i)], rows[slot], gsem[slot])

    def wait_gather(i, slot):
        pltpu.make_async_copy(
            wte_hbm.at[idx_slice(i)], rows[slot], gsem[slot]).wait()

    def issue_store(i, slot):
        pltpu.async_copy(rows[slot], out_slice(i), ssem[slot])

    def wait_store(i, slot):
        pltpu.make_async_copy(rows[slot], out_slice(i), ssem[slot]).wait()

    def add_wpe(i, slot):
        @plsc.parallel_loop(0, SC_W, 1, unroll=1)
        def _(r):
            for c in range(D_SLICES):
                sl = pl.ds(c * LANES, LANES)
                plsc.addupdate(rows[slot].at[r, sl], wpe_v[r, sl])

    def step(i, slot):
        far = (slot + DEPTH) % NB

        @pl.when(i >= DEPTH)
        def _():
            wait_store(i - DEPTH, far)

        @pl.when(i + DEPTH < STEPS)
        def _():
            issue_gather(i + DEPTH, far)
        wait_gather(i, slot)
        add_wpe(i, slot)
        issue_store(i, slot)

    for p in range(DEPTH):
        issue_gather(jnp.int32(p), p)

    def loop_body(j, _):
        i = UNROLL * j
        for r in range(UNROLL):
            step(i + r, r)
        return _

    lax.fori_loop(0, STEPS // UNROLL, loop_body, None)
    for d in range(DEPTH):
        i = STEPS - DEPTH + d
        wait_store(i, i % NB)


@jax.jit
def _embed(input_ids, wte, wpe):
    mesh = plsc.VectorSubcoreMesh(core_axis_name="c", subcore_axis_name="s")
    return pl.kernel(
        _body,
        out_type=jax.ShapeDtypeStruct((B, S, D), jnp.float32),
        mesh=mesh,
        scratch_types=(
            [pltpu.VMEM((B, SC_W), jnp.int32),
             pltpu.VMEM((SC_W, D), jnp.float32)]
            + [pltpu.VMEM((SC_W, D), jnp.float32)] * NB
            + [pltpu.SemaphoreType.DMA] * (2 * NB + 1)
        ),
    )(input_ids.reshape(-1), wte, wpe)


def kernel(input_ids, attention_mask, wte, wpe):
    del attention_mask
    return _embed(input_ids, wte, wpe)
